# baseline (device time: 21334 ns/iter reference)
import jax
import jax.numpy as jnp
from jax import lax
from jax.experimental import pallas as pl
from jax.experimental.pallas import tpu as pltpu

N_DEV = 16
B, Sq, Hq, Dh = 2, 256, 4, 64
BLK = 64
NQB = Sq // BLK
DM = Hq * Dh
R = Sq // N_DEV
CW = B * DM + B * Hq
BF = jnp.bfloat16
F32 = jnp.float32


def kernel(x, Wq, K_ext, V_ext, Wo):
    d_model = x.shape[-1]

    def body(x_ref, wq_ref, k_ref, v_ref, wo_ref, out_ref,
             comb_tx, comb_rx, ctx_buf,
             s1_send, s1_recv, s2_send, s2_recv):
        me = lax.axis_index("i")

        bsem = pltpu.get_barrier_semaphore()
        pl.semaphore_signal(bsem, inc=1, device_id=(me,),
                            device_id_type=pl.DeviceIdType.MESH)
        pl.semaphore_wait(bsem, 1)

        import os
        _scope = (jax.named_scope if os.environ.get("KPROF")
                  else (lambda name: __import__("contextlib").nullcontext()))
        ri = lax.broadcasted_iota(jnp.int32, (Sq, Sq), 0) // BLK
        ci = lax.broadcasted_iota(jnp.int32, (Sq, Sq), 1) // BLK
        bd_mask = ri == ci
        with _scope("compute_partial"):
         for b in range(B):
            q_full = jnp.dot(x_ref[b].astype(BF), wq_ref[...].astype(BF),
                             preferred_element_type=F32)
            for h in range(Hq):
                cols = slice(h * Dh, (h + 1) * Dh)
                qh = q_full[:, cols].astype(BF)
                kh = k_ref[b, :, h, :].astype(BF)
                vh = v_ref[b, :, h, :].astype(BF)
                s = jnp.dot(qh, kh.T,
                            preferred_element_type=F32) * 0.125
                e = jnp.where(bd_mask, jnp.exp(s), 0.0)
                acc = jnp.dot(e.astype(BF), vh,
                              preferred_element_type=F32)
                comb_tx[:, b * DM + h * Dh:b * DM + (h + 1) * Dh] = (
                    acc.astype(BF))
                lcol = B * DM + b * Hq + h
                comb_tx[:, lcol:lcol + 1] = jnp.sum(
                    e, axis=1, keepdims=True).astype(BF)

        p1 = []
        with _scope("p1_issue"):
         for j in range(1, N_DEV):
            tgt = lax.rem(me + j, N_DEV)
            r = pltpu.make_async_remote_copy(
                src_ref=comb_tx.at[pl.ds(tgt * R, R), :],
                dst_ref=comb_rx.at[j - 1],
                send_sem=s1_send.at[j - 1], recv_sem=s1_recv.at[j - 1],
                device_id=(tgt,), device_id_type=pl.DeviceIdType.MESH)
            r.start()
            p1.append(r)

        slc = comb_tx[pl.ds(me * R, R), :].astype(F32)
        with _scope("p1_wait"):
         for r in p1:
            r.wait_recv()
        with _scope("reduce_norm"):
         for j in range(N_DEV - 1):
            slc = slc + comb_rx[j].astype(F32)

         for b in range(B):
            parts = []
            for h in range(Hq):
                a = slc[:, b * DM + h * Dh:b * DM + (h + 1) * Dh]
                lv = slc[:, B * DM + b * Hq + h]
                parts.append(a / lv[:, None])
            ctx_buf[b, pl.ds(me * R, R), :] = jnp.concatenate(
                parts, axis=1).astype(BF)

        p2 = []
        with _scope("p2_issue"):
         for j in range(1, N_DEV):
            tgt = lax.rem(me + j, N_DEV)
            r = pltpu.make_async_remote_copy(
                src_ref=ctx_buf.at[:, pl.ds(me * R, R), :],
                dst_ref=ctx_buf.at[:, pl.ds(me * R, R), :],
                send_sem=s2_send.at[j - 1], recv_sem=s2_recv.at[j - 1],
                device_id=(tgt,), device_id_type=pl.DeviceIdType.MESH)
            r.start()
            p2.append(r)
        with _scope("p2_wait"):
         for r in p2:
            r.wait_recv()
         for r in p1:
            r.wait_send()
         for r in p2:
            r.wait_send()

        with _scope("wo_proj"):
         for b in range(B):
            out_ref[b] = jnp.dot(ctx_buf[b].astype(BF),
                                 wo_ref[...].astype(BF),
                                 preferred_element_type=F32)

    return pl.pallas_call(
        body,
        out_shape=jax.ShapeDtypeStruct((B, Sq, d_model), F32),
        in_specs=[pl.BlockSpec(memory_space=pltpu.VMEM)] * 5,
        out_specs=pl.BlockSpec(memory_space=pltpu.VMEM),
        scratch_shapes=[
            pltpu.VMEM((Sq, CW), BF),
            pltpu.VMEM((N_DEV - 1, R, CW), BF),
            pltpu.VMEM((B, Sq, DM), BF),
            pltpu.SemaphoreType.DMA((N_DEV - 1,)),
            pltpu.SemaphoreType.DMA((N_DEV - 1,)),
            pltpu.SemaphoreType.DMA((N_DEV - 1,)),
            pltpu.SemaphoreType.DMA((N_DEV - 1,)),
        ],
        compiler_params=pltpu.CompilerParams(collective_id=0),
    )(x, Wq, K_ext, V_ext, Wo)


# device time: 19522 ns/iter; 1.0928x vs baseline; 1.0928x over previous
import jax
import jax.numpy as jnp
from jax import lax
from jax.experimental import pallas as pl
from jax.experimental.pallas import tpu as pltpu

N_DEV = 16
B, Sq, Hq, Dh = 2, 256, 4, 64
BLK = 64
NQB = Sq // BLK
DM = Hq * Dh
R = Sq // N_DEV
CW = B * DM + B * Hq
BF = jnp.bfloat16
F32 = jnp.float32


def kernel(x, Wq, K_ext, V_ext, Wo):
    d_model = x.shape[-1]
    K2 = K_ext.reshape(B, Sq, DM)
    V2 = V_ext.reshape(B, Sq, DM)

    def body(x_ref, wq_ref, k_ref, v_ref, wo_ref, out_ref,
             comb_tx, comb_rx, ctx_buf,
             s1_send, s1_recv, s2_send, s2_recv):
        me = lax.axis_index("i")

        bsem = pltpu.get_barrier_semaphore()
        pl.semaphore_signal(bsem, inc=1, device_id=(me,),
                            device_id_type=pl.DeviceIdType.MESH)
        pl.semaphore_wait(bsem, 1)

        import os
        _scope = (jax.named_scope if os.environ.get("KPROF")
                  else (lambda name: __import__("contextlib").nullcontext()))
        ri = lax.broadcasted_iota(jnp.int32, (Sq, Sq), 0) // BLK
        ci = lax.broadcasted_iota(jnp.int32, (Sq, Sq), 1) // BLK
        bd_mask = ri == ci
        with _scope("compute_partial"):
         for b in range(B):
            q_full = jnp.dot(x_ref[b].astype(BF), wq_ref[...].astype(BF),
                             preferred_element_type=F32)
            for h in range(Hq):
                cols = slice(h * Dh, (h + 1) * Dh)
                qh = q_full[:, cols].astype(BF)
                kh = k_ref[b, :, cols].astype(BF)
                vh = v_ref[b, :, cols].astype(BF)
                s = jnp.dot(qh, kh.T,
                            preferred_element_type=F32) * 0.125
                e = jnp.where(bd_mask, jnp.exp(s), 0.0)
                acc = jnp.dot(e.astype(BF), vh,
                              preferred_element_type=F32)
                comb_tx[:, b * DM + h * Dh:b * DM + (h + 1) * Dh] = (
                    acc.astype(BF))
                lcol = B * DM + b * Hq + h
                comb_tx[:, lcol:lcol + 1] = jnp.sum(
                    e, axis=1, keepdims=True).astype(BF)

        p1 = []
        with _scope("p1_issue"):
         for j in range(1, N_DEV):
            tgt = lax.rem(me + j, N_DEV)
            r = pltpu.make_async_remote_copy(
                src_ref=comb_tx.at[pl.ds(tgt * R, R), :],
                dst_ref=comb_rx.at[j - 1],
                send_sem=s1_send.at[j - 1], recv_sem=s1_recv.at[j - 1],
                device_id=(tgt,), device_id_type=pl.DeviceIdType.MESH)
            r.start()
            p1.append(r)

        slc = comb_tx[pl.ds(me * R, R), :].astype(F32)
        with _scope("p1_wait"):
         for r in p1:
            r.wait_recv()
        with _scope("reduce_norm"):
         for j in range(N_DEV - 1):
            slc = slc + comb_rx[j].astype(F32)

         for b in range(B):
            parts = []
            for h in range(Hq):
                a = slc[:, b * DM + h * Dh:b * DM + (h + 1) * Dh]
                lv = slc[:, B * DM + b * Hq + h]
                parts.append(a / lv[:, None])
            ctx_buf[b, pl.ds(me * R, R), :] = jnp.concatenate(
                parts, axis=1).astype(BF)

        p2 = []
        with _scope("p2_issue"):
         for j in range(1, N_DEV):
            tgt = lax.rem(me + j, N_DEV)
            r = pltpu.make_async_remote_copy(
                src_ref=ctx_buf.at[:, pl.ds(me * R, R), :],
                dst_ref=ctx_buf.at[:, pl.ds(me * R, R), :],
                send_sem=s2_send.at[j - 1], recv_sem=s2_recv.at[j - 1],
                device_id=(tgt,), device_id_type=pl.DeviceIdType.MESH)
            r.start()
            p2.append(r)
        with _scope("p2_wait"):
         for r in p2:
            r.wait_recv()
         for r in p1:
            r.wait_send()
         for r in p2:
            r.wait_send()

        with _scope("wo_proj"):
         for b in range(B):
            out_ref[b] = jnp.dot(ctx_buf[b].astype(BF),
                                 wo_ref[...].astype(BF),
                                 preferred_element_type=F32)

    return pl.pallas_call(
        body,
        out_shape=jax.ShapeDtypeStruct((B, Sq, d_model), F32),
        in_specs=[pl.BlockSpec(memory_space=pltpu.VMEM)] * 5,
        out_specs=pl.BlockSpec(memory_space=pltpu.VMEM),
        scratch_shapes=[
            pltpu.VMEM((Sq, CW), BF),
            pltpu.VMEM((N_DEV - 1, R, CW), BF),
            pltpu.VMEM((B, Sq, DM), BF),
            pltpu.SemaphoreType.DMA((N_DEV - 1,)),
            pltpu.SemaphoreType.DMA((N_DEV - 1,)),
            pltpu.SemaphoreType.DMA((N_DEV - 1,)),
            pltpu.SemaphoreType.DMA((N_DEV - 1,)),
        ],
        compiler_params=pltpu.CompilerParams(collective_id=0),
    )(x, Wq, K2, V2, Wo)
